# Initial kernel scaffold; baseline (speedup 1.0000x reference)
import jax
import jax.numpy as jnp
from jax.experimental import pallas as pl


def kernel(node_embeddings, adjacency_lists, W1, b1, W2, b2, W3, b3, W4, b4):
    raise NotImplementedError("write your pallas kernel here")


# speedup vs baseline: 7.1679x; 7.1679x over previous
import functools

import jax
import jax.numpy as jnp
from jax import lax
from jax.experimental import pallas as pl
from jax.experimental.pallas import tpu as pltpu
from jax.experimental.pallas import tpu_sc as plsc

N = 10000
E = 320000
DIM = 128
NUM_CLASSES = 64

NC = 2
NS = 16
NW = NC * NS
EDGES_PER_TILE = E // NW
CHUNK = 80
CHUNKS = EDGES_PER_TILE // CHUNK
NPAD = 10240
ROWS_PER_SUB = NPAD // NS

_sc_mesh = plsc.VectorSubcoreMesh(core_axis_name="c", subcore_axis_name="s")


@functools.partial(
    pl.kernel,
    mesh=_sc_mesh,
    out_type=jax.ShapeDtypeStruct((NC, NPAD, DIM), jnp.float32),
    scratch_types=[
        pltpu.VMEM((CHUNKS, CHUNK), jnp.int32),
        pltpu.VMEM((CHUNKS, CHUNK), jnp.int32),
        pltpu.VMEM((CHUNK, DIM), jnp.float32),
        pltpu.VMEM_SHARED((NPAD, DIM), jnp.float32),
        pltpu.SemaphoreType.DMA,
    ],
)
def _sc_agg(x_hbm, src_hbm, dst_hbm, zeros_hbm, out_hbm,
            src_v, dst_v, rows_v, acc, sem):
    c = lax.axis_index("c")
    s = lax.axis_index("s")
    tile = c * NS + s
    pltpu.sync_copy(src_hbm.at[tile], src_v)
    pltpu.sync_copy(dst_hbm.at[tile], dst_v)
    pltpu.sync_copy(zeros_hbm.at[pl.ds(s * ROWS_PER_SUB, ROWS_PER_SUB)],
                    acc.at[pl.ds(s * ROWS_PER_SUB, ROWS_PER_SUB)])
    plsc.subcore_barrier()

    def body(j, carry):
        pltpu.async_copy(x_hbm.at[src_v.at[j]], rows_v, sem).wait()
        pltpu.sync_copy(rows_v, acc.at[dst_v.at[j]], add=True)
        return carry

    lax.fori_loop(0, CHUNKS, body, 0)
    plsc.subcore_barrier()
    pltpu.sync_copy(acc.at[pl.ds(s * ROWS_PER_SUB, ROWS_PER_SUB)],
                    out_hbm.at[c, pl.ds(s * ROWS_PER_SUB, ROWS_PER_SUB)])


ROW_BLOCK = 1000


def _mlp1_body(x_ref, p_ref, W1_ref, b1_ref, W2_ref, b2_ref, o_ref):
    h = x_ref[...] + p_ref[0] + p_ref[1]
    t = jnp.maximum(
        jnp.dot(h, W1_ref[...], preferred_element_type=jnp.float32) + b1_ref[...],
        0.0)
    y = jnp.dot(t, W2_ref[...], preferred_element_type=jnp.float32) + b2_ref[...]
    o_ref[...] = jnp.maximum(y, 0.0)


def _mlp2_body(x_ref, p_ref, W3_ref, b3_ref, W4_ref, b4_ref, o_ref):
    h = x_ref[...] + p_ref[0] + p_ref[1]
    t = jnp.maximum(
        jnp.dot(h, W3_ref[...], preferred_element_type=jnp.float32) + b3_ref[...],
        0.0)
    z = jnp.dot(t, W4_ref[...], preferred_element_type=jnp.float32) + b4_ref[...]
    z = z - jnp.max(z, axis=-1, keepdims=True)
    ez = jnp.exp(z)
    o_ref[...] = ez / jnp.sum(ez, axis=-1, keepdims=True)


def _mlp1(x, p, W1, b1, W2, b2):
    grid = (N // ROW_BLOCK,)
    return pl.pallas_call(
        _mlp1_body,
        grid=grid,
        in_specs=[
            pl.BlockSpec((ROW_BLOCK, DIM), lambda i: (i, 0)),
            pl.BlockSpec((NC, ROW_BLOCK, DIM), lambda i: (0, i, 0)),
            pl.BlockSpec((DIM, DIM), lambda i: (0, 0)),
            pl.BlockSpec((1, DIM), lambda i: (0, 0)),
            pl.BlockSpec((DIM, DIM), lambda i: (0, 0)),
            pl.BlockSpec((1, DIM), lambda i: (0, 0)),
        ],
        out_specs=pl.BlockSpec((ROW_BLOCK, DIM), lambda i: (i, 0)),
        out_shape=jax.ShapeDtypeStruct((N, DIM), jnp.float32),
    )(x, p, W1, b1, W2, b2)


def _mlp2(x, p, W3, b3, W4, b4):
    grid = (N // ROW_BLOCK,)
    return pl.pallas_call(
        _mlp2_body,
        grid=grid,
        in_specs=[
            pl.BlockSpec((ROW_BLOCK, DIM), lambda i: (i, 0)),
            pl.BlockSpec((NC, ROW_BLOCK, DIM), lambda i: (0, i, 0)),
            pl.BlockSpec((DIM, NUM_CLASSES), lambda i: (0, 0)),
            pl.BlockSpec((1, NUM_CLASSES), lambda i: (0, 0)),
            pl.BlockSpec((NUM_CLASSES, NUM_CLASSES), lambda i: (0, 0)),
            pl.BlockSpec((1, NUM_CLASSES), lambda i: (0, 0)),
        ],
        out_specs=pl.BlockSpec((ROW_BLOCK, NUM_CLASSES), lambda i: (i, 0)),
        out_shape=jax.ShapeDtypeStruct((N, NUM_CLASSES), jnp.float32),
    )(x, p, W3, b3, W4, b4)


def kernel(node_embeddings, adjacency_lists, W1, b1, W2, b2, W3, b3, W4, b4):
    x = node_embeddings.astype(jnp.float32)
    adj = adjacency_lists.astype(jnp.int32)
    src3 = adj[0].reshape(NW, CHUNKS, CHUNK)
    dst3 = adj[1].reshape(NW, CHUNKS, CHUNK)
    zeros = jnp.zeros((NPAD, DIM), jnp.float32)

    p1 = _sc_agg(x, src3, dst3, zeros)
    x1 = _mlp1(x, p1, W1, b1.reshape(1, DIM), W2, b2.reshape(1, DIM))
    p2 = _sc_agg(x1, src3, dst3, zeros)
    return _mlp2(x1, p2, W3, b3.reshape(1, NUM_CLASSES),
                 W4, b4.reshape(1, NUM_CLASSES))
